# baseline (speedup 1.0000x reference)
import jax
import jax.numpy as jnp
from jax.experimental import pallas as pl
from jax.experimental.pallas import tpu as pltpu
from jax.experimental.pallas import tpu_sc as plsc

NUM_EXPERTS = 16
NUM_K = 2
CAP = 1536
RT_BLK = 512
FF_BLK = 256
CHUNK = 512
N_CHUNKS = 2
SC_WIN = 128


def _router_body(n_tokens, x_ref, rw_ref, slots_ref, probs_ref, counts_ref,
                 aux_ref, xq_ref, cnt_scr, psum_scr):
    step = pl.program_id(0)

    @pl.when(step == 0)
    def _():
        cnt_scr[...] = jnp.zeros_like(cnt_scr)
        psum_scr[...] = jnp.zeros_like(psum_scr)

    x = x_ref[...]
    rw = rw_ref[...]
    xq_ref[...] = _act_quant(x).astype(jnp.bfloat16)
    logits = jax.lax.dot_general(
        x.astype(jnp.bfloat16), rw.astype(jnp.bfloat16),
        (((1,), (1,)), ((), ())), preferred_element_type=jnp.float32)
    m = jnp.max(logits, axis=-1, keepdims=True)
    ex = jnp.exp(logits - m)
    probs = ex / jnp.sum(ex, axis=-1, keepdims=True)

    lane = jax.lax.broadcasted_iota(jnp.int32, probs.shape, 1)
    p0 = jnp.max(probs, axis=-1, keepdims=True)
    a0 = jnp.min(jnp.where(probs == p0, lane, NUM_EXPERTS), axis=-1,
                 keepdims=True)
    oh0 = (lane == a0).astype(jnp.float32)
    masked = jnp.where(lane == a0, -jnp.inf, probs)
    p1 = jnp.max(masked, axis=-1, keepdims=True)
    a1 = jnp.min(jnp.where(masked == p1, lane, NUM_EXPERTS), axis=-1,
                 keepdims=True)
    oh1 = (lane == a1).astype(jnp.float32)

    denom = p0 + p1 + 1e-8
    probs_ref[...] = jnp.concatenate([p0 / denom, p1 / denom], axis=1)

    r_iota = jax.lax.broadcasted_iota(jnp.int32, (RT_BLK, RT_BLK), 0)
    c_iota = jax.lax.broadcasted_iota(jnp.int32, (RT_BLK, RT_BLK), 1)
    lt = (c_iota < r_iota).astype(jnp.float32)
    rank0 = jax.lax.dot_general(lt, oh0, (((1,), (0,)), ((), ())),
                                preferred_element_type=jnp.float32)
    rank1 = jax.lax.dot_general(lt, oh1, (((1,), (0,)), ((), ())),
                                preferred_element_type=jnp.float32)
    cnt_row = cnt_scr[...]
    col0 = jnp.sum(oh0, axis=0, keepdims=True)
    col1 = jnp.sum(oh1, axis=0, keepdims=True)
    pos0 = jnp.sum((rank0 + cnt_row) * oh0, axis=-1, keepdims=True)
    pos1 = jnp.sum((rank1 + cnt_row + col0) * oh1, axis=-1, keepdims=True)
    pos0 = jnp.minimum(pos0.astype(jnp.int32), CAP - 1)
    pos1 = jnp.minimum(pos1.astype(jnp.int32), CAP - 1)
    slot0 = a0 * CAP + pos0
    slot1 = a1 * CAP + pos1
    c_off = jax.lax.broadcasted_iota(jnp.int32, (RT_BLK, N_CHUNKS), 1)
    slots_ref[...] = jnp.concatenate(
        [slot0 * N_CHUNKS + c_off, slot1 * N_CHUNKS + c_off], axis=1)

    new_cnt = cnt_row + col0 + col1
    new_psum = psum_scr[...] + jnp.sum(probs, axis=0, keepdims=True)
    cnt_scr[...] = new_cnt
    psum_scr[...] = new_psum

    counts_ref[...] = new_cnt.astype(jnp.int32)
    f = new_cnt / float(n_tokens * NUM_K)
    pbar = new_psum / float(n_tokens)
    aux_ref[...] = (NUM_EXPERTS * jnp.sum(f * pbar)).reshape(1, 1)


def _run_router(x_flat, router_w):
    n_tokens, d = x_flat.shape
    grid = (n_tokens // RT_BLK,)
    return pl.pallas_call(
        lambda *refs: _router_body(n_tokens, *refs),
        grid=grid,
        in_specs=[
            pl.BlockSpec((RT_BLK, d), lambda i: (i, 0)),
            pl.BlockSpec((NUM_EXPERTS, d), lambda i: (0, 0)),
        ],
        out_specs=[
            pl.BlockSpec((RT_BLK, NUM_K * N_CHUNKS), lambda i: (i, 0)),
            pl.BlockSpec((RT_BLK, NUM_K), lambda i: (i, 0)),
            pl.BlockSpec((1, NUM_EXPERTS), lambda i: (0, 0)),
            pl.BlockSpec((1, 1), lambda i: (0, 0)),
            pl.BlockSpec((RT_BLK, d), lambda i: (i, 0)),
        ],
        out_shape=[
            jax.ShapeDtypeStruct((n_tokens, NUM_K * N_CHUNKS), jnp.int32),
            jax.ShapeDtypeStruct((n_tokens, NUM_K), jnp.float32),
            jax.ShapeDtypeStruct((1, NUM_EXPERTS), jnp.int32),
            jax.ShapeDtypeStruct((1, 1), jnp.float32),
            jax.ShapeDtypeStruct((n_tokens, d), jnp.bfloat16),
        ],
        scratch_shapes=[
            pltpu.VMEM((1, NUM_EXPERTS), jnp.float32),
            pltpu.VMEM((1, NUM_EXPERTS), jnp.float32),
        ],
    )(x_flat, router_w)


def _sc_mesh():
    return plsc.VectorSubcoreMesh(core_axis_name="core",
                                  subcore_axis_name="subcore")


def _run_scatter(x_chunks, slots_c):
    n_chunks_tot, cw = x_chunks.shape
    n_win = n_chunks_tot // SC_WIN

    @pl.kernel(out_type=jax.ShapeDtypeStruct(
                   (NUM_EXPERTS * CAP * N_CHUNKS, cw), x_chunks.dtype),
               mesh=_sc_mesh(), scratch_types=[])
    def scatter_kernel(x_hbm, i_hbm, buf_hbm):
        def body(x_vmem, i_vmem):
            pltpu.sync_copy(x_vmem, buf_hbm.at[i_vmem.at[0]])

        pltpu.emit_pipeline(
            body,
            grid=(NUM_K, n_win),
            in_specs=[
                pl.BlockSpec((SC_WIN, cw), index_map=lambda k, i: (i, 0)),
                pl.BlockSpec((1, SC_WIN), index_map=lambda k, i: (k, i)),
            ],
            out_specs=[],
            core_axis_name=("core", "subcore"),
            dimension_semantics=(pltpu.PARALLEL, pltpu.PARALLEL),
        )(x_hbm, i_hbm)

    return scatter_kernel(x_chunks, slots_c)


def _run_gather(y_chunks, slots_c):
    _, cw = y_chunks.shape
    n_k, n_chunks_tot = slots_c.shape
    n_win = n_chunks_tot // SC_WIN

    @pl.kernel(out_type=jax.ShapeDtypeStruct((n_k * n_chunks_tot, cw),
                                             y_chunks.dtype),
               mesh=_sc_mesh(), scratch_types=[])
    def gather_kernel(y_hbm, i_hbm, g_hbm):
        def body(i_vmem, o_vmem):
            pltpu.sync_copy(y_hbm.at[i_vmem.at[0]], o_vmem)

        pltpu.emit_pipeline(
            body,
            grid=(NUM_K, n_win),
            in_specs=[
                pl.BlockSpec((1, SC_WIN), index_map=lambda k, i: (k, i)),
            ],
            out_specs=[
                pl.BlockSpec((SC_WIN, cw),
                             index_map=lambda k, i: (k * n_win + i, 0)),
            ],
            core_axis_name=("core", "subcore"),
            dimension_semantics=(pltpu.PARALLEL, pltpu.PARALLEL),
        )(i_hbm, g_hbm)

    return gather_kernel(y_chunks, slots_c)


def _ternary(w, scale):
    thr = 0.5 * scale
    return jnp.where(w > thr, 1.0, jnp.where(w < -thr, -1.0, 0.0))


def _act_quant(a):
    s = jnp.maximum(jnp.max(jnp.abs(a), axis=-1, keepdims=True), 1e-5)
    return jnp.clip(jnp.round(a * 127.0 / s), -128.0, 127.0) * (s / 127.0)


def _ffn_body(counts_ref, buf_ref, w1_ref, w2_ref, y_ref, qw1_scr, qw2_scr):
    e = pl.program_id(0)
    b = pl.program_id(1)

    @pl.when(b == 0)
    def _():
        w1 = w1_ref[0]
        s1 = jnp.mean(jnp.abs(w1)) + 1e-8
        qw1_scr[...] = (_ternary(w1, s1) * s1).astype(jnp.bfloat16)
        w2 = w2_ref[0]
        s2 = jnp.mean(jnp.abs(w2)) + 1e-8
        qw2_scr[...] = (_ternary(w2, s2) * s2).astype(jnp.bfloat16)

    @pl.when(b * FF_BLK < counts_ref[e])
    def _():
        a = buf_ref[0]
        mm1 = jnp.dot(a, qw1_scr[...], preferred_element_type=jnp.float32)
        r = jnp.square(jnp.maximum(mm1, 0.0))
        h = _act_quant(r)
        y_ref[0] = jnp.dot(h.astype(jnp.bfloat16), qw2_scr[...],
                           preferred_element_type=jnp.float32
                           ).astype(jnp.bfloat16)


def _run_ffn(counts, buf, w1, w2):
    e, cap, d = buf.shape
    f = w1.shape[2]
    grid_spec = pltpu.PrefetchScalarGridSpec(
        num_scalar_prefetch=1,
        grid=(e, cap // FF_BLK),
        in_specs=[
            pl.BlockSpec((1, FF_BLK, d), lambda ei, bi, *_: (ei, bi, 0)),
            pl.BlockSpec((1, d, f), lambda ei, bi, *_: (ei, 0, 0)),
            pl.BlockSpec((1, f, d), lambda ei, bi, *_: (ei, 0, 0)),
        ],
        out_specs=pl.BlockSpec((1, FF_BLK, d), lambda ei, bi, *_: (ei, bi, 0)),
        scratch_shapes=[
            pltpu.VMEM((d, f), jnp.bfloat16),
            pltpu.VMEM((f, d), jnp.bfloat16),
        ],
    )
    return pl.pallas_call(
        _ffn_body,
        grid_spec=grid_spec,
        out_shape=jax.ShapeDtypeStruct((e, cap, d), jnp.bfloat16),
    )(counts, buf, w1, w2)


def _combine_body(g_ref, p_ref, out_ref):
    g0 = g_ref[0].astype(jnp.float32)
    g1 = g_ref[1].astype(jnp.float32)
    p = p_ref[...]
    out_ref[...] = g0 * p[:, 0:1] + g1 * p[:, 1:2]


def _run_combine(g, probs):
    _, n_tokens, d = g.shape
    return pl.pallas_call(
        _combine_body,
        grid=(n_tokens // RT_BLK,),
        in_specs=[
            pl.BlockSpec((NUM_K, RT_BLK, d), lambda i: (0, i, 0)),
            pl.BlockSpec((RT_BLK, NUM_K), lambda i: (i, 0)),
        ],
        out_specs=pl.BlockSpec((RT_BLK, d), lambda i: (i, 0)),
        out_shape=jax.ShapeDtypeStruct((n_tokens, d), jnp.float32),
    )(g, probs)


def kernel(x, router_w, w1, w2):
    bx, tx, d = x.shape
    x_flat = x.reshape(-1, d)
    n_tokens = x_flat.shape[0]

    slots, probs, counts, aux, xq = _run_router(x_flat, router_w)
    slots_c = (slots.reshape(n_tokens, NUM_K, N_CHUNKS)
               .transpose(1, 0, 2).reshape(NUM_K, n_tokens * N_CHUNKS))
    x_chunks = jax.lax.bitcast_convert_type(
        xq.reshape(n_tokens * N_CHUNKS, CHUNK // 2, 2), jnp.int32)
    buf_i = _run_scatter(x_chunks, slots_c)
    buf = jax.lax.bitcast_convert_type(buf_i, jnp.bfloat16).reshape(
        NUM_EXPERTS, CAP, d)
    y = _run_ffn(counts.reshape(NUM_EXPERTS), buf, w1, w2)
    y_chunks = jax.lax.bitcast_convert_type(
        y.reshape(NUM_EXPERTS * CAP * N_CHUNKS, CHUNK // 2, 2), jnp.int32)
    g_i = _run_gather(y_chunks, slots_c)
    g = jax.lax.bitcast_convert_type(g_i, jnp.bfloat16).reshape(
        NUM_K, n_tokens, d)
    out_flat = _run_combine(g, probs)
    return out_flat.reshape(bx, tx, d), aux.reshape(())


# speedup vs baseline: 27.0890x; 27.0890x over previous
import jax
import jax.numpy as jnp
from jax.experimental import pallas as pl
from jax.experimental.pallas import tpu as pltpu
from jax.experimental.pallas import tpu_sc as plsc

NUM_EXPERTS = 16
NUM_K = 2
CAP = 1536
RT_BLK = 512
FF_BLK = 256
CHUNK = 256
N_CHUNKS = 4
SC_WIN = 128


def _router_body(n_tokens, x_ref, rw_ref, slots_ref, probs_ref, counts_ref,
                 aux_ref, xq_ref, cnt_scr, psum_scr):
    step = pl.program_id(0)

    @pl.when(step == 0)
    def _():
        cnt_scr[...] = jnp.zeros_like(cnt_scr)
        psum_scr[...] = jnp.zeros_like(psum_scr)

    x = x_ref[...]
    rw = rw_ref[...]
    xq_ref[...] = _act_quant(x)
    logits = jax.lax.dot_general(
        x.astype(jnp.bfloat16), rw.astype(jnp.bfloat16),
        (((1,), (1,)), ((), ())), preferred_element_type=jnp.float32)
    m = jnp.max(logits, axis=-1, keepdims=True)
    ex = jnp.exp(logits - m)
    probs = ex / jnp.sum(ex, axis=-1, keepdims=True)

    lane = jax.lax.broadcasted_iota(jnp.int32, probs.shape, 1)
    p0 = jnp.max(probs, axis=-1, keepdims=True)
    a0 = jnp.min(jnp.where(probs == p0, lane, NUM_EXPERTS), axis=-1,
                 keepdims=True)
    oh0 = (lane == a0).astype(jnp.float32)
    masked = jnp.where(lane == a0, -jnp.inf, probs)
    p1 = jnp.max(masked, axis=-1, keepdims=True)
    a1 = jnp.min(jnp.where(masked == p1, lane, NUM_EXPERTS), axis=-1,
                 keepdims=True)
    oh1 = (lane == a1).astype(jnp.float32)

    denom = p0 + p1 + 1e-8
    probs_ref[...] = jnp.concatenate([p0 / denom, p1 / denom], axis=1)

    r_iota = jax.lax.broadcasted_iota(jnp.int32, (RT_BLK, RT_BLK), 0)
    c_iota = jax.lax.broadcasted_iota(jnp.int32, (RT_BLK, RT_BLK), 1)
    lt = (c_iota < r_iota).astype(jnp.float32)
    rank0 = jax.lax.dot_general(lt, oh0, (((1,), (0,)), ((), ())),
                                preferred_element_type=jnp.float32)
    rank1 = jax.lax.dot_general(lt, oh1, (((1,), (0,)), ((), ())),
                                preferred_element_type=jnp.float32)
    cnt_row = cnt_scr[...]
    col0 = jnp.sum(oh0, axis=0, keepdims=True)
    col1 = jnp.sum(oh1, axis=0, keepdims=True)
    pos0 = jnp.sum((rank0 + cnt_row) * oh0, axis=-1, keepdims=True)
    pos1 = jnp.sum((rank1 + cnt_row + col0) * oh1, axis=-1, keepdims=True)
    pos0 = jnp.minimum(pos0.astype(jnp.int32), CAP - 1)
    pos1 = jnp.minimum(pos1.astype(jnp.int32), CAP - 1)
    slot0 = a0 * CAP + pos0
    slot1 = a1 * CAP + pos1
    c_off = jax.lax.broadcasted_iota(jnp.int32, (RT_BLK, N_CHUNKS), 1)
    slots_ref[...] = jnp.concatenate(
        [slot0 * N_CHUNKS + c_off, slot1 * N_CHUNKS + c_off], axis=1)

    new_cnt = cnt_row + col0 + col1
    new_psum = psum_scr[...] + jnp.sum(probs, axis=0, keepdims=True)
    cnt_scr[...] = new_cnt
    psum_scr[...] = new_psum

    counts_ref[...] = new_cnt.astype(jnp.int32)
    f = new_cnt / float(n_tokens * NUM_K)
    pbar = new_psum / float(n_tokens)
    aux_ref[...] = (NUM_EXPERTS * jnp.sum(f * pbar)).reshape(1, 1)


def _run_router(x_flat, router_w):
    n_tokens, d = x_flat.shape
    grid = (n_tokens // RT_BLK,)
    return pl.pallas_call(
        lambda *refs: _router_body(n_tokens, *refs),
        grid=grid,
        in_specs=[
            pl.BlockSpec((RT_BLK, d), lambda i: (i, 0)),
            pl.BlockSpec((NUM_EXPERTS, d), lambda i: (0, 0)),
        ],
        out_specs=[
            pl.BlockSpec((RT_BLK, NUM_K * N_CHUNKS), lambda i: (i, 0)),
            pl.BlockSpec((RT_BLK, NUM_K), lambda i: (i, 0)),
            pl.BlockSpec((1, NUM_EXPERTS), lambda i: (0, 0)),
            pl.BlockSpec((1, 1), lambda i: (0, 0)),
            pl.BlockSpec((RT_BLK, d), lambda i: (i, 0)),
        ],
        out_shape=[
            jax.ShapeDtypeStruct((n_tokens, NUM_K * N_CHUNKS), jnp.int32),
            jax.ShapeDtypeStruct((n_tokens, NUM_K), jnp.float32),
            jax.ShapeDtypeStruct((1, NUM_EXPERTS), jnp.int32),
            jax.ShapeDtypeStruct((1, 1), jnp.float32),
            jax.ShapeDtypeStruct((n_tokens, d), jnp.float32),
        ],
        scratch_shapes=[
            pltpu.VMEM((1, NUM_EXPERTS), jnp.float32),
            pltpu.VMEM((1, NUM_EXPERTS), jnp.float32),
        ],
    )(x_flat, router_w)


def _sc_mesh():
    return plsc.VectorSubcoreMesh(core_axis_name="core",
                                  subcore_axis_name="subcore")


def _run_scatter(x_chunks, slots_c):
    n_chunks_tot, cw = x_chunks.shape
    n_win = n_chunks_tot // SC_WIN

    @pl.kernel(out_type=jax.ShapeDtypeStruct(
                   (NUM_EXPERTS * CAP * N_CHUNKS, cw), x_chunks.dtype),
               mesh=_sc_mesh(), scratch_types=[])
    def scatter_kernel(x_hbm, i_hbm, buf_hbm):
        def body(x_vmem, i_vmem):
            pltpu.sync_copy(x_vmem, buf_hbm.at[i_vmem.at[0]])

        pltpu.emit_pipeline(
            body,
            grid=(NUM_K, n_win),
            in_specs=[
                pl.BlockSpec((SC_WIN, cw), index_map=lambda k, i: (i, 0)),
                pl.BlockSpec((1, SC_WIN), index_map=lambda k, i: (k, i)),
            ],
            out_specs=[],
            core_axis_name=("core", "subcore"),
            dimension_semantics=(pltpu.PARALLEL, pltpu.PARALLEL),
        )(x_hbm, i_hbm)

    return scatter_kernel(x_chunks, slots_c)


def _run_gather(y_chunks, slots_c):
    _, cw = y_chunks.shape
    n_k, n_chunks_tot = slots_c.shape
    n_win = n_chunks_tot // SC_WIN

    @pl.kernel(out_type=jax.ShapeDtypeStruct((n_k * n_chunks_tot, cw),
                                             y_chunks.dtype),
               mesh=_sc_mesh(), scratch_types=[])
    def gather_kernel(y_hbm, i_hbm, g_hbm):
        def body(i_vmem, o_vmem):
            pltpu.sync_copy(y_hbm.at[i_vmem.at[0]], o_vmem)

        pltpu.emit_pipeline(
            body,
            grid=(NUM_K, n_win),
            in_specs=[
                pl.BlockSpec((1, SC_WIN), index_map=lambda k, i: (k, i)),
            ],
            out_specs=[
                pl.BlockSpec((SC_WIN, cw),
                             index_map=lambda k, i: (k * n_win + i, 0)),
            ],
            core_axis_name=("core", "subcore"),
            dimension_semantics=(pltpu.PARALLEL, pltpu.PARALLEL),
        )(i_hbm, g_hbm)

    return gather_kernel(y_chunks, slots_c)


def _ternary(w, scale):
    thr = 0.5 * scale
    return jnp.where(w > thr, 1.0, jnp.where(w < -thr, -1.0, 0.0))


def _act_quant(a):
    s = jnp.maximum(jnp.max(jnp.abs(a), axis=-1, keepdims=True), 1e-5)
    return jnp.clip(jnp.round(a * 127.0 / s), -128.0, 127.0) * (s / 127.0)


def _ffn_body(counts_ref, buf_ref, w1_ref, w2_ref, y_ref, qw1_scr, qw2_scr):
    e = pl.program_id(0)
    b = pl.program_id(1)

    @pl.when(b == 0)
    def _():
        w1 = w1_ref[0]
        s1 = jnp.mean(jnp.abs(w1)) + 1e-8
        qw1_scr[...] = (_ternary(w1, s1) * s1).astype(jnp.bfloat16)
        w2 = w2_ref[0]
        s2 = jnp.mean(jnp.abs(w2)) + 1e-8
        qw2_scr[...] = (_ternary(w2, s2) * s2).astype(jnp.bfloat16)

    @pl.when(b * FF_BLK < counts_ref[e])
    def _():
        a = buf_ref[0]
        mm1 = jnp.dot(a.astype(jnp.bfloat16), qw1_scr[...],
                      preferred_element_type=jnp.float32)
        r = jnp.square(jnp.maximum(mm1, 0.0))
        h = _act_quant(r)
        y_ref[0] = jnp.dot(h.astype(jnp.bfloat16), qw2_scr[...],
                           preferred_element_type=jnp.float32)


def _run_ffn(counts, buf, w1, w2):
    e, cap, d = buf.shape
    f = w1.shape[2]
    grid_spec = pltpu.PrefetchScalarGridSpec(
        num_scalar_prefetch=1,
        grid=(e, cap // FF_BLK),
        in_specs=[
            pl.BlockSpec((1, FF_BLK, d), lambda ei, bi, *_: (ei, bi, 0)),
            pl.BlockSpec((1, d, f), lambda ei, bi, *_: (ei, 0, 0)),
            pl.BlockSpec((1, f, d), lambda ei, bi, *_: (ei, 0, 0)),
        ],
        out_specs=pl.BlockSpec((1, FF_BLK, d), lambda ei, bi, *_: (ei, bi, 0)),
        scratch_shapes=[
            pltpu.VMEM((d, f), jnp.bfloat16),
            pltpu.VMEM((f, d), jnp.bfloat16),
        ],
    )
    return pl.pallas_call(
        _ffn_body,
        grid_spec=grid_spec,
        out_shape=jax.ShapeDtypeStruct((e, cap, d), jnp.float32),
    )(counts, buf, w1, w2)


def _combine_body(g_ref, p_ref, out_ref):
    g0 = g_ref[0].astype(jnp.float32)
    g1 = g_ref[1].astype(jnp.float32)
    p = p_ref[...]
    out_ref[...] = g0 * p[:, 0:1] + g1 * p[:, 1:2]


def _run_combine(g, probs):
    _, n_tokens, d = g.shape
    return pl.pallas_call(
        _combine_body,
        grid=(n_tokens // RT_BLK,),
        in_specs=[
            pl.BlockSpec((NUM_K, RT_BLK, d), lambda i: (0, i, 0)),
            pl.BlockSpec((RT_BLK, NUM_K), lambda i: (i, 0)),
        ],
        out_specs=pl.BlockSpec((RT_BLK, d), lambda i: (i, 0)),
        out_shape=jax.ShapeDtypeStruct((n_tokens, d), jnp.float32),
    )(g, probs)


def kernel(x, router_w, w1, w2):
    bx, tx, d = x.shape
    x_flat = x.reshape(-1, d)
    n_tokens = x_flat.shape[0]

    slots, probs, counts, aux, xq = _run_router(x_flat, router_w)
    slots_c = (slots.reshape(n_tokens, NUM_K, N_CHUNKS)
               .transpose(1, 0, 2).reshape(NUM_K, n_tokens * N_CHUNKS))
    x_chunks = xq.reshape(n_tokens * N_CHUNKS, CHUNK)
    buf = _run_scatter(x_chunks, slots_c)
    y = _run_ffn(counts.reshape(NUM_EXPERTS),
                 buf.reshape(NUM_EXPERTS, CAP, d), w1, w2)
    g = _run_gather(y.reshape(NUM_EXPERTS * CAP * N_CHUNKS, CHUNK), slots_c)
    out_flat = _run_combine(g.reshape(NUM_K, n_tokens, d), probs)
    return out_flat.reshape(bx, tx, d), aux.reshape(())


# speedup vs baseline: 27.8057x; 1.0265x over previous
import jax
import jax.numpy as jnp
from jax.experimental import pallas as pl
from jax.experimental.pallas import tpu as pltpu
from jax.experimental.pallas import tpu_sc as plsc

NUM_EXPERTS = 16
NUM_K = 2
CAP = 1536
RT_BLK = 512
FF_BLK = 512
CHUNK = 256
N_CHUNKS = 4
SC_WIN = 128


def _router_body(n_tokens, x_ref, rw_ref, slots_ref, probs_ref, counts_ref,
                 aux_ref, xq_ref, cnt_scr, psum_scr):
    step = pl.program_id(0)

    @pl.when(step == 0)
    def _():
        cnt_scr[...] = jnp.zeros_like(cnt_scr)
        psum_scr[...] = jnp.zeros_like(psum_scr)

    x = x_ref[...]
    rw = rw_ref[...]
    xq_ref[...] = _act_quant(x)
    logits = jax.lax.dot_general(
        x.astype(jnp.bfloat16), rw.astype(jnp.bfloat16),
        (((1,), (1,)), ((), ())), preferred_element_type=jnp.float32)
    m = jnp.max(logits, axis=-1, keepdims=True)
    ex = jnp.exp(logits - m)
    probs = ex / jnp.sum(ex, axis=-1, keepdims=True)

    lane = jax.lax.broadcasted_iota(jnp.int32, probs.shape, 1)
    p0 = jnp.max(probs, axis=-1, keepdims=True)
    a0 = jnp.min(jnp.where(probs == p0, lane, NUM_EXPERTS), axis=-1,
                 keepdims=True)
    oh0 = (lane == a0).astype(jnp.float32)
    masked = jnp.where(lane == a0, -jnp.inf, probs)
    p1 = jnp.max(masked, axis=-1, keepdims=True)
    a1 = jnp.min(jnp.where(masked == p1, lane, NUM_EXPERTS), axis=-1,
                 keepdims=True)
    oh1 = (lane == a1).astype(jnp.float32)

    denom = p0 + p1 + 1e-8
    probs_ref[...] = jnp.concatenate([p0 / denom, p1 / denom], axis=1)

    r_iota = jax.lax.broadcasted_iota(jnp.int32, (RT_BLK, RT_BLK), 0)
    c_iota = jax.lax.broadcasted_iota(jnp.int32, (RT_BLK, RT_BLK), 1)
    lt = (c_iota < r_iota).astype(jnp.float32)
    rank0 = jax.lax.dot_general(lt, oh0, (((1,), (0,)), ((), ())),
                                preferred_element_type=jnp.float32)
    rank1 = jax.lax.dot_general(lt, oh1, (((1,), (0,)), ((), ())),
                                preferred_element_type=jnp.float32)
    cnt_row = cnt_scr[...]
    col0 = jnp.sum(oh0, axis=0, keepdims=True)
    col1 = jnp.sum(oh1, axis=0, keepdims=True)
    pos0 = jnp.sum((rank0 + cnt_row) * oh0, axis=-1, keepdims=True)
    pos1 = jnp.sum((rank1 + cnt_row + col0) * oh1, axis=-1, keepdims=True)
    pos0 = jnp.minimum(pos0.astype(jnp.int32), CAP - 1)
    pos1 = jnp.minimum(pos1.astype(jnp.int32), CAP - 1)
    slot0 = a0 * CAP + pos0
    slot1 = a1 * CAP + pos1
    c_off = jax.lax.broadcasted_iota(jnp.int32, (RT_BLK, N_CHUNKS), 1)
    slots_ref[...] = jnp.concatenate(
        [slot0 * N_CHUNKS + c_off, slot1 * N_CHUNKS + c_off], axis=1)

    new_cnt = cnt_row + col0 + col1
    new_psum = psum_scr[...] + jnp.sum(probs, axis=0, keepdims=True)
    cnt_scr[...] = new_cnt
    psum_scr[...] = new_psum

    counts_ref[...] = new_cnt.astype(jnp.int32)
    f = new_cnt / float(n_tokens * NUM_K)
    pbar = new_psum / float(n_tokens)
    aux_ref[...] = (NUM_EXPERTS * jnp.sum(f * pbar)).reshape(1, 1)


def _run_router(x_flat, router_w):
    n_tokens, d = x_flat.shape
    grid = (n_tokens // RT_BLK,)
    return pl.pallas_call(
        lambda *refs: _router_body(n_tokens, *refs),
        grid=grid,
        in_specs=[
            pl.BlockSpec((RT_BLK, d), lambda i: (i, 0)),
            pl.BlockSpec((NUM_EXPERTS, d), lambda i: (0, 0)),
        ],
        out_specs=[
            pl.BlockSpec((RT_BLK, NUM_K * N_CHUNKS), lambda i: (i, 0)),
            pl.BlockSpec((RT_BLK, NUM_K), lambda i: (i, 0)),
            pl.BlockSpec((1, NUM_EXPERTS), lambda i: (0, 0)),
            pl.BlockSpec((1, 1), lambda i: (0, 0)),
            pl.BlockSpec((RT_BLK, d), lambda i: (i, 0)),
        ],
        out_shape=[
            jax.ShapeDtypeStruct((n_tokens, NUM_K * N_CHUNKS), jnp.int32),
            jax.ShapeDtypeStruct((n_tokens, NUM_K), jnp.float32),
            jax.ShapeDtypeStruct((1, NUM_EXPERTS), jnp.int32),
            jax.ShapeDtypeStruct((1, 1), jnp.float32),
            jax.ShapeDtypeStruct((n_tokens, d), jnp.float32),
        ],
        scratch_shapes=[
            pltpu.VMEM((1, NUM_EXPERTS), jnp.float32),
            pltpu.VMEM((1, NUM_EXPERTS), jnp.float32),
        ],
    )(x_flat, router_w)


def _sc_mesh():
    return plsc.VectorSubcoreMesh(core_axis_name="core",
                                  subcore_axis_name="subcore")


def _run_scatter(x_chunks, slots_c):
    n_chunks_tot, cw = x_chunks.shape
    n_win = n_chunks_tot // SC_WIN

    @pl.kernel(out_type=jax.ShapeDtypeStruct(
                   (NUM_EXPERTS * CAP * N_CHUNKS, cw), x_chunks.dtype),
               mesh=_sc_mesh(), scratch_types=[])
    def scatter_kernel(x_hbm, i_hbm, buf_hbm):
        def body(x_vmem, i_vmem):
            pltpu.sync_copy(x_vmem, buf_hbm.at[i_vmem.at[0]])

        pltpu.emit_pipeline(
            body,
            grid=(NUM_K, n_win),
            in_specs=[
                pl.BlockSpec((SC_WIN, cw), index_map=lambda k, i: (i, 0)),
                pl.BlockSpec((1, SC_WIN), index_map=lambda k, i: (k, i)),
            ],
            out_specs=[],
            core_axis_name=("core", "subcore"),
            dimension_semantics=(pltpu.PARALLEL, pltpu.PARALLEL),
        )(x_hbm, i_hbm)

    return scatter_kernel(x_chunks, slots_c)


def _run_gather(y_chunks, slots_c):
    _, cw = y_chunks.shape
    n_k, n_chunks_tot = slots_c.shape
    n_win = n_chunks_tot // SC_WIN

    @pl.kernel(out_type=jax.ShapeDtypeStruct((n_k * n_chunks_tot, cw),
                                             y_chunks.dtype),
               mesh=_sc_mesh(), scratch_types=[])
    def gather_kernel(y_hbm, i_hbm, g_hbm):
        def body(i_vmem, o_vmem):
            pltpu.sync_copy(y_hbm.at[i_vmem.at[0]], o_vmem)

        pltpu.emit_pipeline(
            body,
            grid=(NUM_K, n_win),
            in_specs=[
                pl.BlockSpec((1, SC_WIN), index_map=lambda k, i: (k, i)),
            ],
            out_specs=[
                pl.BlockSpec((SC_WIN, cw),
                             index_map=lambda k, i: (k * n_win + i, 0)),
            ],
            core_axis_name=("core", "subcore"),
            dimension_semantics=(pltpu.PARALLEL, pltpu.PARALLEL),
        )(i_hbm, g_hbm)

    return gather_kernel(y_chunks, slots_c)


def _ternary(w, scale):
    thr = 0.5 * scale
    return jnp.where(w > thr, 1.0, jnp.where(w < -thr, -1.0, 0.0))


def _act_quant(a):
    s = jnp.maximum(jnp.max(jnp.abs(a), axis=-1, keepdims=True), 1e-5)
    return jnp.clip(jnp.round(a * 127.0 / s), -128.0, 127.0) * (s / 127.0)


def _wq_body(w_ref, qw_ref):
    w = w_ref[0]
    s = jnp.mean(jnp.abs(w)) + 1e-8
    qw_ref[0] = (_ternary(w, s) * s).astype(jnp.bfloat16)


def _run_weight_quant(w):
    e, a, b = w.shape
    return pl.pallas_call(
        _wq_body,
        grid=(e,),
        in_specs=[pl.BlockSpec((1, a, b), lambda i: (i, 0, 0))],
        out_specs=pl.BlockSpec((1, a, b), lambda i: (i, 0, 0)),
        out_shape=jax.ShapeDtypeStruct((e, a, b), jnp.bfloat16),
    )(w)


def _ffn_body(counts_ref, buf_ref, qw1_ref, qw2_ref, y_ref):
    e = pl.program_id(0)
    b = pl.program_id(1)

    @pl.when(b * FF_BLK < counts_ref[e])
    def _():
        a = buf_ref[0]
        mm1 = jnp.dot(a.astype(jnp.bfloat16), qw1_ref[0],
                      preferred_element_type=jnp.float32)
        r = jnp.square(jnp.maximum(mm1, 0.0))
        h = _act_quant(r)
        y_ref[0] = jnp.dot(h.astype(jnp.bfloat16), qw2_ref[0],
                           preferred_element_type=jnp.float32)


def _run_ffn(counts, buf, qw1, qw2):
    e, cap, d = buf.shape
    f = qw1.shape[2]
    grid_spec = pltpu.PrefetchScalarGridSpec(
        num_scalar_prefetch=1,
        grid=(e, cap // FF_BLK),
        in_specs=[
            pl.BlockSpec((1, FF_BLK, d), lambda ei, bi, *_: (ei, bi, 0)),
            pl.BlockSpec((1, d, f), lambda ei, bi, *_: (ei, 0, 0)),
            pl.BlockSpec((1, f, d), lambda ei, bi, *_: (ei, 0, 0)),
        ],
        out_specs=pl.BlockSpec((1, FF_BLK, d), lambda ei, bi, *_: (ei, bi, 0)),
    )
    return pl.pallas_call(
        _ffn_body,
        grid_spec=grid_spec,
        out_shape=jax.ShapeDtypeStruct((e, cap, d), jnp.float32),
    )(counts, buf, qw1, qw2)


def _combine_body(g_ref, p_ref, out_ref):
    g0 = g_ref[0].astype(jnp.float32)
    g1 = g_ref[1].astype(jnp.float32)
    p = p_ref[...]
    out_ref[...] = g0 * p[:, 0:1] + g1 * p[:, 1:2]


def _run_combine(g, probs):
    _, n_tokens, d = g.shape
    return pl.pallas_call(
        _combine_body,
        grid=(n_tokens // RT_BLK,),
        in_specs=[
            pl.BlockSpec((NUM_K, RT_BLK, d), lambda i: (0, i, 0)),
            pl.BlockSpec((RT_BLK, NUM_K), lambda i: (i, 0)),
        ],
        out_specs=pl.BlockSpec((RT_BLK, d), lambda i: (i, 0)),
        out_shape=jax.ShapeDtypeStruct((n_tokens, d), jnp.float32),
    )(g, probs)


def kernel(x, router_w, w1, w2):
    bx, tx, d = x.shape
    x_flat = x.reshape(-1, d)
    n_tokens = x_flat.shape[0]

    slots, probs, counts, aux, xq = _run_router(x_flat, router_w)
    slots_c = (slots.reshape(n_tokens, NUM_K, N_CHUNKS)
               .transpose(1, 0, 2).reshape(NUM_K, n_tokens * N_CHUNKS))
    x_chunks = xq.reshape(n_tokens * N_CHUNKS, CHUNK)
    buf = _run_scatter(x_chunks, slots_c)
    qw1 = _run_weight_quant(w1)
    qw2 = _run_weight_quant(w2)
    y = _run_ffn(counts.reshape(NUM_EXPERTS),
                 buf.reshape(NUM_EXPERTS, CAP, d), qw1, qw2)
    g = _run_gather(y.reshape(NUM_EXPERTS * CAP * N_CHUNKS, CHUNK), slots_c)
    out_flat = _run_combine(g.reshape(NUM_K, n_tokens, d), probs)
    return out_flat.reshape(bx, tx, d), aux.reshape(())


# speedup vs baseline: 28.6265x; 1.0295x over previous
import jax
import jax.numpy as jnp
from jax.experimental import pallas as pl
from jax.experimental.pallas import tpu as pltpu
from jax.experimental.pallas import tpu_sc as plsc

NUM_EXPERTS = 16
NUM_K = 2
CAP = 1536
RT_BLK = 512
FF_BLK = 512
CHUNK = 256
N_CHUNKS = 4
SC_WIN = 128


def _router_body(n_tokens, x_ref, rw_ref, slots_ref, probs_ref, counts_ref,
                 aux_ref, xq_ref, cnt_scr, psum_scr):
    step = pl.program_id(0)

    @pl.when(step == 0)
    def _():
        cnt_scr[...] = jnp.zeros_like(cnt_scr)
        psum_scr[...] = jnp.zeros_like(psum_scr)

    x = x_ref[...]
    rw = rw_ref[...]
    xq_ref[...] = _act_quant(x)
    logits = jax.lax.dot_general(
        x.astype(jnp.bfloat16), rw.astype(jnp.bfloat16),
        (((1,), (1,)), ((), ())), preferred_element_type=jnp.float32)
    m = jnp.max(logits, axis=-1, keepdims=True)
    ex = jnp.exp(logits - m)
    probs = ex / jnp.sum(ex, axis=-1, keepdims=True)

    lane = jax.lax.broadcasted_iota(jnp.int32, probs.shape, 1)
    p0 = jnp.max(probs, axis=-1, keepdims=True)
    a0 = jnp.min(jnp.where(probs == p0, lane, NUM_EXPERTS), axis=-1,
                 keepdims=True)
    oh0 = (lane == a0).astype(jnp.float32)
    masked = jnp.where(lane == a0, -jnp.inf, probs)
    p1 = jnp.max(masked, axis=-1, keepdims=True)
    a1 = jnp.min(jnp.where(masked == p1, lane, NUM_EXPERTS), axis=-1,
                 keepdims=True)
    oh1 = (lane == a1).astype(jnp.float32)

    denom = p0 + p1 + 1e-8
    probs_ref[...] = jnp.concatenate([p0 / denom, p1 / denom], axis=1)

    r_iota = jax.lax.broadcasted_iota(jnp.int32, (RT_BLK, RT_BLK), 0)
    c_iota = jax.lax.broadcasted_iota(jnp.int32, (RT_BLK, RT_BLK), 1)
    lt = (c_iota < r_iota).astype(jnp.float32)
    rank0 = jax.lax.dot_general(lt, oh0, (((1,), (0,)), ((), ())),
                                preferred_element_type=jnp.float32)
    rank1 = jax.lax.dot_general(lt, oh1, (((1,), (0,)), ((), ())),
                                preferred_element_type=jnp.float32)
    cnt_row = cnt_scr[...]
    col0 = jnp.sum(oh0, axis=0, keepdims=True)
    col1 = jnp.sum(oh1, axis=0, keepdims=True)
    pos0 = jnp.sum((rank0 + cnt_row) * oh0, axis=-1, keepdims=True)
    pos1 = jnp.sum((rank1 + cnt_row + col0) * oh1, axis=-1, keepdims=True)
    pos0 = jnp.minimum(pos0.astype(jnp.int32), CAP - 1)
    pos1 = jnp.minimum(pos1.astype(jnp.int32), CAP - 1)
    slot0 = a0 * CAP + pos0
    slot1 = a1 * CAP + pos1
    c_off = jax.lax.broadcasted_iota(jnp.int32, (RT_BLK, N_CHUNKS), 1)
    slots_ref[...] = jnp.concatenate(
        [slot0 * N_CHUNKS + c_off, slot1 * N_CHUNKS + c_off], axis=1)

    new_cnt = cnt_row + col0 + col1
    new_psum = psum_scr[...] + jnp.sum(probs, axis=0, keepdims=True)
    cnt_scr[...] = new_cnt
    psum_scr[...] = new_psum

    counts_ref[...] = new_cnt.astype(jnp.int32)
    f = new_cnt / float(n_tokens * NUM_K)
    pbar = new_psum / float(n_tokens)
    aux_ref[...] = (NUM_EXPERTS * jnp.sum(f * pbar)).reshape(1, 1)


def _run_router(x_flat, router_w):
    n_tokens, d = x_flat.shape
    grid = (n_tokens // RT_BLK,)
    return pl.pallas_call(
        lambda *refs: _router_body(n_tokens, *refs),
        grid=grid,
        in_specs=[
            pl.BlockSpec((RT_BLK, d), lambda i: (i, 0)),
            pl.BlockSpec((NUM_EXPERTS, d), lambda i: (0, 0)),
        ],
        out_specs=[
            pl.BlockSpec((RT_BLK, NUM_K * N_CHUNKS), lambda i: (i, 0)),
            pl.BlockSpec((RT_BLK, NUM_K), lambda i: (i, 0)),
            pl.BlockSpec((1, NUM_EXPERTS), lambda i: (0, 0)),
            pl.BlockSpec((1, 1), lambda i: (0, 0)),
            pl.BlockSpec((RT_BLK, d), lambda i: (i, 0)),
        ],
        out_shape=[
            jax.ShapeDtypeStruct((n_tokens, NUM_K * N_CHUNKS), jnp.int32),
            jax.ShapeDtypeStruct((n_tokens, NUM_K), jnp.float32),
            jax.ShapeDtypeStruct((1, NUM_EXPERTS), jnp.int32),
            jax.ShapeDtypeStruct((1, 1), jnp.float32),
            jax.ShapeDtypeStruct((n_tokens, d), jnp.float32),
        ],
        scratch_shapes=[
            pltpu.VMEM((1, NUM_EXPERTS), jnp.float32),
            pltpu.VMEM((1, NUM_EXPERTS), jnp.float32),
        ],
    )(x_flat, router_w)


def _sc_mesh():
    return plsc.VectorSubcoreMesh(core_axis_name="core",
                                  subcore_axis_name="subcore")


def _run_scatter(x_chunks, slots_c):
    n_chunks_tot, cw = x_chunks.shape
    n_win = n_chunks_tot // SC_WIN

    @pl.kernel(out_type=jax.ShapeDtypeStruct(
                   (NUM_EXPERTS * CAP * N_CHUNKS, cw), x_chunks.dtype),
               mesh=_sc_mesh(), scratch_types=[])
    def scatter_kernel(x_hbm, i_hbm, buf_hbm):
        def body(x_vmem, i_vmem):
            pltpu.sync_copy(x_vmem, buf_hbm.at[i_vmem.at[0]])

        pltpu.emit_pipeline(
            body,
            grid=(NUM_K, n_win),
            in_specs=[
                pl.BlockSpec((SC_WIN, cw), index_map=lambda k, i: (i, 0)),
                pl.BlockSpec((1, SC_WIN), index_map=lambda k, i: (k, i)),
            ],
            out_specs=[],
            core_axis_name=("core", "subcore"),
            dimension_semantics=(pltpu.PARALLEL, pltpu.PARALLEL),
        )(x_hbm, i_hbm)

    return scatter_kernel(x_chunks, slots_c)


def _run_gather(y_chunks, slots_c):
    _, cw = y_chunks.shape
    n_k, n_chunks_tot = slots_c.shape
    n_win = n_chunks_tot // SC_WIN

    @pl.kernel(out_type=jax.ShapeDtypeStruct((n_k * n_chunks_tot, cw),
                                             y_chunks.dtype),
               mesh=_sc_mesh(), scratch_types=[])
    def gather_kernel(y_hbm, i_hbm, g_hbm):
        def body(i_vmem, o_vmem):
            pltpu.sync_copy(y_hbm.at[i_vmem.at[0]], o_vmem)

        pltpu.emit_pipeline(
            body,
            grid=(NUM_K, n_win),
            in_specs=[
                pl.BlockSpec((1, SC_WIN), index_map=lambda k, i: (k, i)),
            ],
            out_specs=[
                pl.BlockSpec((SC_WIN, cw),
                             index_map=lambda k, i: (k * n_win + i, 0)),
            ],
            core_axis_name=("core", "subcore"),
            dimension_semantics=(pltpu.PARALLEL, pltpu.PARALLEL),
        )(i_hbm, g_hbm)

    return gather_kernel(y_chunks, slots_c)


def _ternary(w, scale):
    thr = 0.5 * scale
    return jnp.where(w > thr, 1.0, jnp.where(w < -thr, -1.0, 0.0))


def _act_quant(a):
    s = jnp.maximum(jnp.max(jnp.abs(a), axis=-1, keepdims=True), 1e-5)
    return jnp.clip(jnp.round(a * 127.0 / s), -128.0, 127.0) * (s / 127.0)


def _wq_body(w1_ref, w2_ref, qw1_ref, qw2_ref):
    w1 = w1_ref[0]
    s1 = jnp.mean(jnp.abs(w1)) + 1e-8
    qw1_ref[0] = (_ternary(w1, s1) * s1).astype(jnp.bfloat16)
    w2 = w2_ref[0]
    s2 = jnp.mean(jnp.abs(w2)) + 1e-8
    qw2_ref[0] = (_ternary(w2, s2) * s2).astype(jnp.bfloat16)


def _run_weight_quant(w1, w2):
    e, d, f = w1.shape
    return pl.pallas_call(
        _wq_body,
        grid=(e,),
        in_specs=[pl.BlockSpec((1, d, f), lambda i: (i, 0, 0)),
                  pl.BlockSpec((1, f, d), lambda i: (i, 0, 0))],
        out_specs=[pl.BlockSpec((1, d, f), lambda i: (i, 0, 0)),
                   pl.BlockSpec((1, f, d), lambda i: (i, 0, 0))],
        out_shape=[jax.ShapeDtypeStruct((e, d, f), jnp.bfloat16),
                   jax.ShapeDtypeStruct((e, f, d), jnp.bfloat16)],
    )(w1, w2)


def _ffn_body(counts_ref, buf_ref, qw1_ref, qw2_ref, y_ref):
    e = pl.program_id(0)
    b = pl.program_id(1)

    @pl.when(b * FF_BLK < counts_ref[e])
    def _():
        a = buf_ref[0]
        mm1 = jnp.dot(a.astype(jnp.bfloat16), qw1_ref[0],
                      preferred_element_type=jnp.float32)
        r = jnp.square(jnp.maximum(mm1, 0.0))
        h = _act_quant(r)
        y_ref[0] = jnp.dot(h.astype(jnp.bfloat16), qw2_ref[0],
                           preferred_element_type=jnp.float32)


def _run_ffn(counts, buf, qw1, qw2):
    e, cap, d = buf.shape
    f = qw1.shape[2]
    grid_spec = pltpu.PrefetchScalarGridSpec(
        num_scalar_prefetch=1,
        grid=(e, cap // FF_BLK),
        in_specs=[
            pl.BlockSpec((1, FF_BLK, d), lambda ei, bi, *_: (ei, bi, 0)),
            pl.BlockSpec((1, d, f), lambda ei, bi, *_: (ei, 0, 0)),
            pl.BlockSpec((1, f, d), lambda ei, bi, *_: (ei, 0, 0)),
        ],
        out_specs=pl.BlockSpec((1, FF_BLK, d), lambda ei, bi, *_: (ei, bi, 0)),
    )
    return pl.pallas_call(
        _ffn_body,
        grid_spec=grid_spec,
        out_shape=jax.ShapeDtypeStruct((e, cap, d), jnp.float32),
    )(counts, buf, qw1, qw2)


def _combine_body(g_ref, p_ref, out_ref):
    g0 = g_ref[0].astype(jnp.float32)
    g1 = g_ref[1].astype(jnp.float32)
    p = p_ref[...]
    out_ref[...] = g0 * p[:, 0:1] + g1 * p[:, 1:2]


def _run_combine(g, probs):
    _, n_tokens, d = g.shape
    return pl.pallas_call(
        _combine_body,
        grid=(n_tokens // RT_BLK,),
        in_specs=[
            pl.BlockSpec((NUM_K, RT_BLK, d), lambda i: (0, i, 0)),
            pl.BlockSpec((RT_BLK, NUM_K), lambda i: (i, 0)),
        ],
        out_specs=pl.BlockSpec((RT_BLK, d), lambda i: (i, 0)),
        out_shape=jax.ShapeDtypeStruct((n_tokens, d), jnp.float32),
    )(g, probs)


def kernel(x, router_w, w1, w2):
    bx, tx, d = x.shape
    x_flat = x.reshape(-1, d)
    n_tokens = x_flat.shape[0]

    slots, probs, counts, aux, xq = _run_router(x_flat, router_w)
    slots_c = (slots.reshape(n_tokens, NUM_K, N_CHUNKS)
               .transpose(1, 0, 2).reshape(NUM_K, n_tokens * N_CHUNKS))
    x_chunks = xq.reshape(n_tokens * N_CHUNKS, CHUNK)
    buf = _run_scatter(x_chunks, slots_c)
    qw1, qw2 = _run_weight_quant(w1, w2)
    y = _run_ffn(counts.reshape(NUM_EXPERTS),
                 buf.reshape(NUM_EXPERTS, CAP, d), qw1, qw2)
    g = _run_gather(y.reshape(NUM_EXPERTS * CAP * N_CHUNKS, CHUNK), slots_c)
    out_flat = _run_combine(g.reshape(NUM_K, n_tokens, d), probs)
    return out_flat.reshape(bx, tx, d), aux.reshape(())
